# baseline (speedup 1.0000x reference)
import jax
import jax.numpy as jnp
from jax.experimental import pallas as pl
from jax.experimental.pallas import tpu as pltpu

H, W, A = 100, 152, 3
N_ANCHORS = H * W * A
PRE_NMS_TOPK = 2000
POST_NMS_TOPK = 1000
NMS_THRESH = 0.7
IMG_H, IMG_W = 800.0, 1216.0

M_PAD = 2048
CHUNK = 128
N_CHUNKS = M_PAD // CHUNK

_INTERPRET = False


def _nms_body(boxes_ref, boxesT_ref, out_ref, q_ref, kvec_ref):
    x1r = boxesT_ref[0:1, :]
    y1r = boxesT_ref[1:2, :]
    x2r = boxesT_ref[2:3, :]
    y2r = boxesT_ref[3:4, :]
    area_r = (x2r - x1r) * (y2r - y1r)

    def build(c, carry):
        b = boxes_ref[pl.ds(c * CHUNK, CHUNK), :]
        x1i = b[:, 0:1]
        y1i = b[:, 1:2]
        x2i = b[:, 2:3]
        y2i = b[:, 3:4]
        area_i = (x2i - x1i) * (y2i - y1i)
        wx = jnp.clip(jnp.minimum(x2i, x2r) - jnp.maximum(x1i, x1r), 0.0)
        wy = jnp.clip(jnp.minimum(y2i, y2r) - jnp.maximum(y1i, y1r), 0.0)
        inter = wx * wy
        iou = inter / (area_i + area_r - inter + 1e-9)
        ag = c * CHUNK + jax.lax.broadcasted_iota(jnp.int32, (CHUNK, M_PAD), 0)
        bg = jax.lax.broadcasted_iota(jnp.int32, (CHUNK, M_PAD), 1)
        q_ref[pl.ds(c * CHUNK, CHUNK), :] = (
            (iou > NMS_THRESH) & (ag < bg)).astype(jnp.float32)
        return carry

    jax.lax.fori_loop(0, N_CHUNKS, build, 0)

    valid = (jax.lax.broadcasted_iota(jnp.int32, (1, M_PAD), 1)
             < PRE_NMS_TOPK)
    k0 = valid.astype(jnp.float32)

    def cond(carry):
        return carry[1]

    def body(carry):
        k, _ = carry
        cnt = jnp.dot(k, q_ref[...], preferred_element_type=jnp.float32)
        k_new = jnp.where((cnt == 0.0) & valid, 1.0, 0.0)
        return k_new, jnp.any(k_new != k)

    k, _ = jax.lax.while_loop(cond, body, (k0, jnp.bool_(True)))
    kvec_ref[...] = k

    out_ref[...] = jnp.zeros((POST_NMS_TOPK, 4), jnp.float32)

    def compact(c, carry):
        ag = jax.lax.broadcasted_iota(jnp.int32, (M_PAD, CHUNK), 0)
        ig = c * CHUNK + jax.lax.broadcasted_iota(jnp.int32, (M_PAD, CHUNK), 1)
        lcol = (ag < ig).astype(jnp.float32)
        slot = jnp.dot(k, lcol, preferred_element_type=jnp.float32)
        kc = kvec_ref[:, pl.ds(c * CHUNK, CHUNK)]
        rr = jax.lax.broadcasted_iota(jnp.int32, (POST_NMS_TOPK, CHUNK), 0)
        slot_i = slot.astype(jnp.int32)
        pt = ((slot_i == rr) & (kc == 1.0)).astype(jnp.float32)
        bc = boxes_ref[pl.ds(c * CHUNK, CHUNK), :]
        out_ref[...] += jnp.dot(pt, bc, preferred_element_type=jnp.float32)
        return carry

    jax.lax.fori_loop(0, N_CHUNKS, compact, 0)


def _nms_compact(boxes):
    boxes_pad = jnp.concatenate(
        [boxes, jnp.zeros((M_PAD - PRE_NMS_TOPK, 4), jnp.float32)], axis=0)
    boxes_t = boxes_pad.T
    return pl.pallas_call(
        _nms_body,
        out_shape=jax.ShapeDtypeStruct((POST_NMS_TOPK, 4), jnp.float32),
        scratch_shapes=[pltpu.VMEM((M_PAD, M_PAD), jnp.float32),
                        pltpu.VMEM((1, M_PAD), jnp.float32)],
        interpret=_INTERPRET,
    )(boxes_pad, boxes_t)


def _conv2d(x, w, b, padding):
    y = jax.lax.conv_general_dilated(
        x, w, (1, 1), padding, dimension_numbers=('NCHW', 'OIHW', 'NCHW'))
    return y + b[None, :, None, None]


def _decode(anchors, deltas):
    w = anchors[:, 2] - anchors[:, 0]
    h = anchors[:, 3] - anchors[:, 1]
    cx = anchors[:, 0] + 0.5 * w
    cy = anchors[:, 1] + 0.5 * h
    dx, dy = deltas[:, 0], deltas[:, 1]
    dw = jnp.clip(deltas[:, 2], -4.0, 4.0)
    dh = jnp.clip(deltas[:, 3], -4.0, 4.0)
    pcx = dx * w + cx
    pcy = dy * h + cy
    pw = jnp.exp(dw) * w
    ph = jnp.exp(dh) * h
    return jnp.stack([pcx - 0.5 * pw, pcy - 0.5 * ph,
                      pcx + 0.5 * pw, pcy + 0.5 * ph], axis=-1)


def kernel(features, conv_w, conv_b, obj_w, obj_b, delta_w, delta_b, anchors):
    x = jax.nn.relu(_conv2d(features, conv_w, conv_b, 'SAME'))
    s = _conv2d(x, obj_w, obj_b, 'VALID')
    d = _conv2d(x, delta_w, delta_b, 'VALID')
    bs = s.shape[0]
    scores = jnp.transpose(s, (0, 2, 3, 1)).reshape(bs, H * W * A)[0]
    deltas = jnp.transpose(
        d.reshape(bs, A, 4, H, W), (0, 3, 4, 1, 2)).reshape(bs, H * W * A, 4)[0]
    top_scores, top_idx = jax.lax.top_k(scores, PRE_NMS_TOPK)
    props = _decode(anchors[top_idx], deltas[top_idx])
    props = jnp.stack([
        jnp.clip(props[:, 0], 0.0, IMG_W),
        jnp.clip(props[:, 1], 0.0, IMG_H),
        jnp.clip(props[:, 2], 0.0, IMG_W),
        jnp.clip(props[:, 3], 0.0, IMG_H),
    ], axis=-1)
    return props
    return _nms_compact(props)


# speedup vs baseline: 1.1199x; 1.1199x over previous
import jax
import jax.numpy as jnp
from jax.experimental import pallas as pl
from jax.experimental.pallas import tpu as pltpu

H, W, A = 100, 152, 3
N_ANCHORS = H * W * A
PRE_NMS_TOPK = 2000
POST_NMS_TOPK = 1000
NMS_THRESH = 0.7
IMG_H, IMG_W = 800.0, 1216.0

M_PAD = 2048
CHUNK = 128
N_CHUNKS = M_PAD // CHUNK

HP, WP = H + 2, W + 2
P_VALID = HP * WP
T_CONV = 512
N_T = (P_VALID + T_CONV - 1) // T_CONV
P_PAD = N_T * T_CONV
MARGIN = WP + 1
X_ROWS = ((P_PAD + 2 * MARGIN + T_CONV - 1) // T_CONV) * T_CONV
WIDE = ((T_CONV + 2 * MARGIN + 511) // 512) * 512
_OFFS = tuple((dh - 1) * WP + (dw - 1) + MARGIN
              for dh in range(3) for dw in range(3))

_INTERPRET = False


def _conv_body(x_ref, w_ref, wh_ref, bc_ref, bh_ref, o_ref):
    t = pl.program_id(0)
    xw = x_ref[pl.ds(t * T_CONV, WIDE), :]
    acc = jnp.zeros((T_CONV, 256), jnp.float32)
    for k in range(9):
        off = _OFFS[k]
        acc += jnp.dot(xw[off:off + T_CONV, :], w_ref[k],
                       preferred_element_type=jnp.float32)
    xr = jax.nn.relu(acc + bc_ref[...])
    o_ref[...] = jnp.dot(xr, wh_ref[...],
                         preferred_element_type=jnp.float32) + bh_ref[...]


def _conv_head(features, conv_w, conv_b, obj_w, obj_b, delta_w, delta_b):
    xp = jnp.pad(features[0], ((0, 0), (1, 1), (1, 1)))
    xp = xp.reshape(256, P_VALID).T
    xb = jnp.zeros((X_ROWS, 256), jnp.float32)
    xb = jax.lax.dynamic_update_slice(xb, xp, (MARGIN, 0))
    w9 = jnp.transpose(conv_w, (2, 3, 1, 0)).reshape(9, 256, 256)
    wh = jnp.concatenate([obj_w[:, :, 0, 0], delta_w[:, :, 0, 0]], axis=0).T
    bh = jnp.concatenate([obj_b, delta_b])[None, :]
    out = pl.pallas_call(
        _conv_body,
        grid=(N_T,),
        in_specs=[
            pl.BlockSpec((X_ROWS, 256), lambda t: (0, 0)),
            pl.BlockSpec((9, 256, 256), lambda t: (0, 0, 0)),
            pl.BlockSpec((256, 15), lambda t: (0, 0)),
            pl.BlockSpec((1, 256), lambda t: (0, 0)),
            pl.BlockSpec((1, 15), lambda t: (0, 0)),
        ],
        out_specs=pl.BlockSpec((T_CONV, 15), lambda t: (t, 0)),
        out_shape=jax.ShapeDtypeStruct((P_PAD, 15), jnp.float32),
        interpret=_INTERPRET,
    )(xb, w9, wh, conv_b[None, :], bh)
    hw = out[:P_VALID].reshape(HP, WP, 15)[1:1 + H, 1:1 + W]
    scores = hw[..., :A].reshape(N_ANCHORS)
    deltas = hw[..., A:].reshape(H, W, A, 4).reshape(N_ANCHORS, 4)
    return scores, deltas


def _nms_body(boxes_ref, boxesT_ref, out_ref, q_ref, kvec_ref):
    x1r = boxesT_ref[0:1, :]
    y1r = boxesT_ref[1:2, :]
    x2r = boxesT_ref[2:3, :]
    y2r = boxesT_ref[3:4, :]
    area_r = (x2r - x1r) * (y2r - y1r)

    def build(c, carry):
        b = boxes_ref[pl.ds(c * CHUNK, CHUNK), :]
        x1i = b[:, 0:1]
        y1i = b[:, 1:2]
        x2i = b[:, 2:3]
        y2i = b[:, 3:4]
        area_i = (x2i - x1i) * (y2i - y1i)
        wx = jnp.clip(jnp.minimum(x2i, x2r) - jnp.maximum(x1i, x1r), 0.0)
        wy = jnp.clip(jnp.minimum(y2i, y2r) - jnp.maximum(y1i, y1r), 0.0)
        inter = wx * wy
        iou = inter / (area_i + area_r - inter + 1e-9)
        ag = c * CHUNK + jax.lax.broadcasted_iota(jnp.int32, (CHUNK, M_PAD), 0)
        bg = jax.lax.broadcasted_iota(jnp.int32, (CHUNK, M_PAD), 1)
        q_ref[pl.ds(c * CHUNK, CHUNK), :] = (
            (iou > NMS_THRESH) & (ag < bg)).astype(jnp.float32)
        return carry

    jax.lax.fori_loop(0, N_CHUNKS, build, 0)

    valid = (jax.lax.broadcasted_iota(jnp.int32, (1, M_PAD), 1)
             < PRE_NMS_TOPK)
    k0 = valid.astype(jnp.float32)

    def cond(carry):
        return carry[1]

    def body(carry):
        k, _ = carry
        cnt = jnp.dot(k, q_ref[...], preferred_element_type=jnp.float32)
        k_new = jnp.where((cnt == 0.0) & valid, 1.0, 0.0)
        return k_new, jnp.any(k_new != k)

    k, _ = jax.lax.while_loop(cond, body, (k0, jnp.bool_(True)))
    kvec_ref[...] = k

    out_ref[...] = jnp.zeros((POST_NMS_TOPK, 4), jnp.float32)

    def compact(c, carry):
        ag = jax.lax.broadcasted_iota(jnp.int32, (M_PAD, CHUNK), 0)
        ig = c * CHUNK + jax.lax.broadcasted_iota(jnp.int32, (M_PAD, CHUNK), 1)
        lcol = (ag < ig).astype(jnp.float32)
        slot = jnp.dot(k, lcol, preferred_element_type=jnp.float32)
        kc = kvec_ref[:, pl.ds(c * CHUNK, CHUNK)]
        rr = jax.lax.broadcasted_iota(jnp.int32, (POST_NMS_TOPK, CHUNK), 0)
        slot_i = slot.astype(jnp.int32)
        pt = ((slot_i == rr) & (kc == 1.0)).astype(jnp.float32)
        bc = boxes_ref[pl.ds(c * CHUNK, CHUNK), :]
        out_ref[...] += jnp.dot(pt, bc, preferred_element_type=jnp.float32)
        return carry

    jax.lax.fori_loop(0, N_CHUNKS, compact, 0)


def _nms_compact(boxes):
    boxes_pad = jnp.concatenate(
        [boxes, jnp.zeros((M_PAD - PRE_NMS_TOPK, 4), jnp.float32)], axis=0)
    boxes_t = boxes_pad.T
    return pl.pallas_call(
        _nms_body,
        out_shape=jax.ShapeDtypeStruct((POST_NMS_TOPK, 4), jnp.float32),
        scratch_shapes=[pltpu.VMEM((M_PAD, M_PAD), jnp.float32),
                        pltpu.VMEM((1, M_PAD), jnp.float32)],
        interpret=_INTERPRET,
    )(boxes_pad, boxes_t)


def _decode(anchors, deltas):
    w = anchors[:, 2] - anchors[:, 0]
    h = anchors[:, 3] - anchors[:, 1]
    cx = anchors[:, 0] + 0.5 * w
    cy = anchors[:, 1] + 0.5 * h
    dx, dy = deltas[:, 0], deltas[:, 1]
    dw = jnp.clip(deltas[:, 2], -4.0, 4.0)
    dh = jnp.clip(deltas[:, 3], -4.0, 4.0)
    pcx = dx * w + cx
    pcy = dy * h + cy
    pw = jnp.exp(dw) * w
    ph = jnp.exp(dh) * h
    return jnp.stack([pcx - 0.5 * pw, pcy - 0.5 * ph,
                      pcx + 0.5 * pw, pcy + 0.5 * ph], axis=-1)


def kernel(features, conv_w, conv_b, obj_w, obj_b, delta_w, delta_b, anchors):
    scores, deltas = _conv_head(
        features, conv_w, conv_b, obj_w, obj_b, delta_w, delta_b)
    top_scores, top_idx = jax.lax.top_k(scores, PRE_NMS_TOPK)
    props = _decode(anchors[top_idx], deltas[top_idx])
    props = jnp.stack([
        jnp.clip(props[:, 0], 0.0, IMG_W),
        jnp.clip(props[:, 1], 0.0, IMG_H),
        jnp.clip(props[:, 2], 0.0, IMG_W),
        jnp.clip(props[:, 3], 0.0, IMG_H),
    ], axis=-1)
    return _nms_compact(props)


# speedup vs baseline: 1.1661x; 1.0413x over previous
import jax
import jax.numpy as jnp
from jax.experimental import pallas as pl
from jax.experimental.pallas import tpu as pltpu

H, W, A = 100, 152, 3
N_ANCHORS = H * W * A
PRE_NMS_TOPK = 2000
POST_NMS_TOPK = 1000
NMS_THRESH = 0.7
IMG_H, IMG_W = 800.0, 1216.0

M_PAD = 2048
CHUNK = 128
N_CHUNKS = M_PAD // CHUNK

HP, WP = H + 2, W + 2
P_VALID = HP * WP
T_CONV = 512
N_T = (P_VALID + T_CONV - 1) // T_CONV
P_PAD = N_T * T_CONV
MARGIN = WP + 1
X_ROWS = ((P_PAD + 2 * MARGIN + T_CONV - 1) // T_CONV) * T_CONV
WIDE = ((T_CONV + 2 * MARGIN + 511) // 512) * 512
_OFFS = tuple((dh - 1) * WP + (dw - 1) + MARGIN
              for dh in range(3) for dw in range(3))

_INTERPRET = False


def _conv_body(x_ref, w_ref, wh_ref, bc_ref, bh_ref, o_ref):
    t = pl.program_id(0)
    xw = x_ref[pl.ds(t * T_CONV, WIDE), :]
    acc = jnp.zeros((T_CONV, 256), jnp.float32)
    for k in range(9):
        off = _OFFS[k]
        acc += jnp.dot(xw[off:off + T_CONV, :], w_ref[k],
                       preferred_element_type=jnp.float32)
    xr = jax.nn.relu(acc + bc_ref[...])
    o_ref[...] = jnp.dot(xr, wh_ref[...],
                         preferred_element_type=jnp.float32) + bh_ref[...]


def _conv_head(features, conv_w, conv_b, obj_w, obj_b, delta_w, delta_b):
    xp = jnp.pad(features[0], ((0, 0), (1, 1), (1, 1)))
    xp = xp.reshape(256, P_VALID).T
    xb = jnp.zeros((X_ROWS, 256), jnp.float32)
    xb = jax.lax.dynamic_update_slice(xb, xp, (MARGIN, 0))
    w9 = jnp.transpose(conv_w, (2, 3, 1, 0)).reshape(9, 256, 256)
    wh = jnp.concatenate([obj_w[:, :, 0, 0], delta_w[:, :, 0, 0]], axis=0).T
    bh = jnp.concatenate([obj_b, delta_b])[None, :]
    out = pl.pallas_call(
        _conv_body,
        grid=(N_T,),
        in_specs=[
            pl.BlockSpec((X_ROWS, 256), lambda t: (0, 0)),
            pl.BlockSpec((9, 256, 256), lambda t: (0, 0, 0)),
            pl.BlockSpec((256, 15), lambda t: (0, 0)),
            pl.BlockSpec((1, 256), lambda t: (0, 0)),
            pl.BlockSpec((1, 15), lambda t: (0, 0)),
        ],
        out_specs=pl.BlockSpec((T_CONV, 15), lambda t: (t, 0)),
        out_shape=jax.ShapeDtypeStruct((P_PAD, 15), jnp.float32),
        interpret=_INTERPRET,
    )(xb, w9, wh, conv_b[None, :], bh)
    hw = out[:P_VALID].reshape(HP, WP, 15)[1:1 + H, 1:1 + W]
    scores = hw[..., :A].reshape(N_ANCHORS)
    deltas = hw[..., A:].reshape(H, W, A, 4).reshape(N_ANCHORS, 4)
    return scores, deltas


def _decode8(a0, a1, a2, a3, d0, d1, d2, d3):
    w = a2 - a0
    h = a3 - a1
    cx = a0 + 0.5 * w
    cy = a1 + 0.5 * h
    dw = jnp.clip(d2, -4.0, 4.0)
    dh = jnp.clip(d3, -4.0, 4.0)
    pcx = d0 * w + cx
    pcy = d1 * h + cy
    pw = jnp.exp(dw) * w
    ph = jnp.exp(dh) * h
    x1 = jnp.clip(pcx - 0.5 * pw, 0.0, IMG_W)
    y1 = jnp.clip(pcy - 0.5 * ph, 0.0, IMG_H)
    x2 = jnp.clip(pcx + 0.5 * pw, 0.0, IMG_W)
    y2 = jnp.clip(pcy + 0.5 * ph, 0.0, IMG_H)
    return x1, y1, x2, y2


def _nms_body(ad_ref, adt_ref, out_ref, q_ref, bx_ref):
    x1r, y1r, x2r, y2r = _decode8(*(adt_ref[i:i + 1, :] for i in range(8)))
    area_r = (x2r - x1r) * (y2r - y1r)

    for c in range(N_CHUNKS):
        ad = ad_ref[c * CHUNK:(c + 1) * CHUNK, :]
        x1i, y1i, x2i, y2i = _decode8(*(ad[:, i:i + 1] for i in range(8)))
        bx_ref[c * CHUNK:(c + 1) * CHUNK, :] = jnp.concatenate(
            [x1i, y1i, x2i, y2i], axis=1)
        area_i = (x2i - x1i) * (y2i - y1i)
        if c > 0:
            q_ref[c * CHUNK:(c + 1) * CHUNK, :c * CHUNK] = jnp.zeros(
                (CHUNK, c * CHUNK), jnp.bfloat16)
        sl = slice(c * CHUNK, (c + 1) * CHUNK)
        wx = jnp.clip(jnp.minimum(x2i, x2r[:, sl]) - jnp.maximum(x1i, x1r[:, sl]), 0.0)
        wy = jnp.clip(jnp.minimum(y2i, y2r[:, sl]) - jnp.maximum(y1i, y1r[:, sl]), 0.0)
        inter = wx * wy
        iou = inter / (area_i + area_r[:, sl] - inter + 1e-9)
        al = jax.lax.broadcasted_iota(jnp.int32, (CHUNK, CHUNK), 0)
        bl = jax.lax.broadcasted_iota(jnp.int32, (CHUNK, CHUNK), 1)
        q_ref[sl, sl] = ((iou > NMS_THRESH) & (al < bl)).astype(jnp.bfloat16)
        if c + 1 < N_CHUNKS:
            sr = slice((c + 1) * CHUNK, M_PAD)
            wx = jnp.clip(jnp.minimum(x2i, x2r[:, sr]) - jnp.maximum(x1i, x1r[:, sr]), 0.0)
            wy = jnp.clip(jnp.minimum(y2i, y2r[:, sr]) - jnp.maximum(y1i, y1r[:, sr]), 0.0)
            inter = wx * wy
            iou = inter / (area_i + area_r[:, sr] - inter + 1e-9)
            q_ref[sl, sr] = (iou > NMS_THRESH).astype(jnp.bfloat16)

    valid = (jax.lax.broadcasted_iota(jnp.int32, (1, M_PAD), 1)
             < PRE_NMS_TOPK)
    k0 = valid.astype(jnp.float32)

    def cond(carry):
        return carry[1]

    def body(carry):
        k, _ = carry
        cnt = jnp.dot(k.astype(jnp.bfloat16), q_ref[...],
                      preferred_element_type=jnp.float32)
        k_new = jnp.where((cnt == 0.0) & valid, 1.0, 0.0)
        return k_new, jnp.any(k_new != k)

    k, _ = jax.lax.while_loop(cond, body, (k0, jnp.bool_(True)))

    u128 = (jax.lax.broadcasted_iota(jnp.int32, (CHUNK, CHUNK), 0)
            < jax.lax.broadcasted_iota(jnp.int32, (CHUNK, CHUNK), 1)
            ).astype(jnp.float32)
    rr = jax.lax.broadcasted_iota(jnp.int32, (POST_NMS_TOPK, CHUNK), 0)
    acc = jnp.zeros((POST_NMS_TOPK, 4), jnp.float32)
    base = jnp.float32(0.0)
    for c in range(N_CHUNKS):
        kc = k[:, c * CHUNK:(c + 1) * CHUNK]
        slot = base + jnp.dot(kc, u128, preferred_element_type=jnp.float32)
        base = base + jnp.sum(kc)
        pt = ((slot.astype(jnp.int32) == rr) & (kc == 1.0)).astype(jnp.float32)
        bc = bx_ref[c * CHUNK:(c + 1) * CHUNK, :]
        acc += jnp.dot(pt, bc, preferred_element_type=jnp.float32)
    out_ref[...] = acc


def _decode_nms_compact(anchors_g, deltas_g):
    ad = jnp.concatenate([anchors_g, deltas_g], axis=1)
    ad = jnp.concatenate(
        [ad, jnp.zeros((M_PAD - PRE_NMS_TOPK, 8), jnp.float32)], axis=0)
    return pl.pallas_call(
        _nms_body,
        out_shape=jax.ShapeDtypeStruct((POST_NMS_TOPK, 4), jnp.float32),
        scratch_shapes=[pltpu.VMEM((M_PAD, M_PAD), jnp.bfloat16),
                        pltpu.VMEM((M_PAD, 4), jnp.float32)],
        interpret=_INTERPRET,
    )(ad, ad.T)


def kernel(features, conv_w, conv_b, obj_w, obj_b, delta_w, delta_b, anchors):
    scores, deltas = _conv_head(
        features, conv_w, conv_b, obj_w, obj_b, delta_w, delta_b)
    top_scores, top_idx = jax.lax.top_k(scores, PRE_NMS_TOPK)
    return _decode_nms_compact(anchors[top_idx], deltas[top_idx])
